# Initial kernel scaffold; baseline (speedup 1.0000x reference)
import jax
import jax.numpy as jnp
from jax.experimental import pallas as pl


def kernel(x, w0, b0, w1, b1, w2, b2, w3, b3):
    raise NotImplementedError("write your pallas kernel here")


# speedup vs baseline: 8.9497x; 8.9497x over previous
import jax
import jax.numpy as jnp
from jax.experimental import pallas as pl
from jax.experimental.pallas import tpu as pltpu

_K = 4


def _kernel_body(x_ref, w0, b0, w1, b1, w2, b2, w3, b3, sel_ref, out_ref,
                 a1e, a1o, a2e, a2o, a3e, a3o):
    f32 = jnp.float32
    bf16 = jnp.bfloat16

    a1o[pl.ds(480, 8), :] = jnp.zeros((8, 32), bf16)
    a2e[pl.ds(112, 8), :] = jnp.zeros((8, 64), bf16)
    a2o[pl.ds(112, 8), :] = jnp.zeros((8, 64), bf16)
    a3e[pl.ds(24, 8), :] = jnp.zeros((8, 128), bf16)
    a3o[pl.ds(24, 8), :] = jnp.zeros((8, 128), bf16)

    acc = None
    for a in range(2):
        for b in range(2):
            t = x_ref[pl.ds(a * 32 + b, 992), :]
            p = jnp.dot(t, w0[a * 2 + b], preferred_element_type=f32)
            acc = p if acc is None else acc + p
    res0 = jnp.maximum(acc + b0[...], 0.0).astype(bf16)
    for yy in range(16):
        a1e[pl.ds(yy * 32, 32), :] = res0[yy * 64:yy * 64 + 32, :]
    for yy in range(15):
        a1o[pl.ds(yy * 32, 32), :] = res0[yy * 64 + 32:yy * 64 + 64, :]

    def layer(in_e, in_o, w_ref, b_ref, pitch, rows_out):
        tt2 = 2 * rows_out
        acc = None
        for p, ref in ((0, in_e), (1, in_o)):
            for a in range(2):
                for kw in range(_K):
                    t = ref[pl.ds(a * pitch + kw, tt2), :]
                    w = w_ref[(2 * a + p) * _K + kw]
                    part = jnp.dot(t, w, preferred_element_type=f32)
                    acc = part if acc is None else acc + part
        sel = sel_ref[0:rows_out, 0:tt2]
        dec = jnp.dot(sel, acc.astype(bf16), preferred_element_type=f32)
        return jnp.maximum(dec + b_ref[...], 0.0).astype(bf16)

    r1 = layer(a1e, a1o, w1, b1, 32, 224)
    for yy in range(7):
        a2e[pl.ds(yy * 16, 16), :] = r1[yy * 32:yy * 32 + 16, :]
        a2o[pl.ds(yy * 16, 16), :] = r1[yy * 32 + 16:yy * 32 + 32, :]

    r2 = layer(a2e, a2o, w2, b2, 16, 48)
    for yy in range(3):
        a3e[pl.ds(yy * 8, 8), :] = r2[yy * 16:yy * 16 + 8, :]
        a3o[pl.ds(yy * 8, 8), :] = r2[yy * 16 + 8:yy * 16 + 16, :]

    r3 = layer(a3e, a3o, w3, b3, 8, 8).astype(f32)
    out_ref[0, :, :] = r3[0:2, :]
    out_ref[1, :, :] = r3[4:6, :]


def kernel(x, w0, b0, w1, b1, w2, b2, w3, b3):
    N, Cin, H, W = x.shape
    bf16 = jnp.bfloat16

    xs = x.reshape(N, Cin, 32, 2, 32, 2)
    xs = jnp.transpose(xs, (0, 2, 4, 3, 5, 1)).reshape(N, 1024, 2 * 2 * Cin)
    xs = jnp.pad(xs, ((0, 0), (0, 8), (0, 0))).astype(bf16)

    co0 = w0.shape[0]
    w0t = jnp.transpose(w0, (2, 3, 1, 0)).reshape(2, 2, 2, 2, Cin, co0)
    w0t = jnp.transpose(w0t, (0, 2, 1, 3, 4, 5)).reshape(4, 2 * 2 * Cin, co0)
    w0t = w0t.astype(bf16)

    def prep(w):
        co, ci, kh, kw = w.shape
        return jnp.transpose(w, (2, 3, 1, 0)).reshape(kh * kw, ci, co).astype(bf16)

    w1t, w2t, w3t = prep(w1), prep(w2), prep(w3)
    b0r = b0.reshape(1, co0)
    b1r = b1.reshape(1, w1.shape[0])
    b2r = b2.reshape(1, w2.shape[0])
    b3r = b3.reshape(1, w3.shape[0])

    rows = jax.lax.broadcasted_iota(jnp.int32, (224, 448), 0)
    cols = jax.lax.broadcasted_iota(jnp.int32, (224, 448), 1)
    sel = (cols == 2 * rows).astype(bf16)

    def bcast(op):
        return pl.BlockSpec(op.shape, lambda n, _nd=len(op.shape): (0,) * _nd)

    operands = [xs, w0t, b0r, w1t, b1r, w2t, b2r, w3t, b3r, sel]
    in_specs = [pl.BlockSpec((None, 1032, 2 * 2 * Cin), lambda n: (n, 0, 0))]
    in_specs += [bcast(op) for op in operands[1:]]

    co3 = w3.shape[0]
    out = pl.pallas_call(
        _kernel_body,
        out_shape=jax.ShapeDtypeStruct((N, 2, 2, co3), jnp.float32),
        grid_spec=pltpu.PrefetchScalarGridSpec(
            num_scalar_prefetch=0,
            grid=(N,),
            in_specs=in_specs,
            out_specs=pl.BlockSpec((None, 2, 2, co3), lambda n: (n, 0, 0, 0)),
            scratch_shapes=[
                pltpu.VMEM((512, 32), bf16),
                pltpu.VMEM((488, 32), bf16),
                pltpu.VMEM((120, 64), bf16),
                pltpu.VMEM((120, 64), bf16),
                pltpu.VMEM((32, 128), bf16),
                pltpu.VMEM((32, 128), bf16),
            ]),
        compiler_params=pltpu.CompilerParams(dimension_semantics=("parallel",)),
        cost_estimate=pl.CostEstimate(
            flops=2 * N * (992 * 4 * 12 * 32 + 448 * 16 * 32 * 64
                           + 96 * 16 * 64 * 128 + 16 * 16 * 128 * 256),
            transcendentals=0,
            bytes_accessed=int(xs.size * 2 + N * 2 * 2 * co3 * 4)),
    )(*operands)

    return jnp.transpose(out, (0, 3, 1, 2))


# speedup vs baseline: 17.4090x; 1.9452x over previous
import jax
import jax.numpy as jnp
from jax.experimental import pallas as pl
from jax.experimental.pallas import tpu as pltpu

_K = 4
_B = 4


def _kernel_body(x_ref, w0, b0, w1, b1, w2, b2, w3, b3, sel_ref, out_ref,
                 a1e, a1o, a2e, a2o, a3e, a3o):
    f32 = jnp.float32
    bf16 = jnp.bfloat16

    for b in range(_B):
        a1o[b, pl.ds(480, 8), :] = jnp.zeros((8, 32), bf16)
        a2e[b, pl.ds(112, 8), :] = jnp.zeros((8, 64), bf16)
        a2o[b, pl.ds(112, 8), :] = jnp.zeros((8, 64), bf16)
        a3e[b, pl.ds(24, 8), :] = jnp.zeros((8, 128), bf16)
        a3o[b, pl.ds(24, 8), :] = jnp.zeros((8, 128), bf16)

    accs = [None] * _B
    for a in range(2):
        for q in range(2):
            w = w0[a * 2 + q]
            for b in range(_B):
                t = x_ref[b, pl.ds(a * 32 + q, 992), :]
                part = jnp.dot(t, w, preferred_element_type=f32)
                accs[b] = part if accs[b] is None else accs[b] + part
    for b in range(_B):
        res0 = jnp.maximum(accs[b] + b0[...], 0.0).astype(bf16)
        for yy in range(16):
            a1e[b, pl.ds(yy * 32, 32), :] = res0[yy * 64:yy * 64 + 32, :]
        for yy in range(15):
            a1o[b, pl.ds(yy * 32, 32), :] = res0[yy * 64 + 32:yy * 64 + 64, :]

    def layer(in_e, in_o, w_ref, b_ref, pitch, rows_out):
        tt2 = 2 * rows_out
        accs = [None] * _B
        for p, ref in ((0, in_e), (1, in_o)):
            for a in range(2):
                for kw in range(_K):
                    w = w_ref[(2 * a + p) * _K + kw]
                    for b in range(_B):
                        t = ref[b, pl.ds(a * pitch + kw, tt2), :]
                        part = jnp.dot(t, w, preferred_element_type=f32)
                        accs[b] = part if accs[b] is None else accs[b] + part
        sel = sel_ref[0:rows_out, 0:tt2]
        outs = []
        for b in range(_B):
            dec = jnp.dot(sel, accs[b].astype(bf16), preferred_element_type=f32)
            outs.append(jnp.maximum(dec + b_ref[...], 0.0).astype(bf16))
        return outs

    r1 = layer(a1e, a1o, w1, b1, 32, 224)
    for b in range(_B):
        for yy in range(7):
            a2e[b, pl.ds(yy * 16, 16), :] = r1[b][yy * 32:yy * 32 + 16, :]
            a2o[b, pl.ds(yy * 16, 16), :] = r1[b][yy * 32 + 16:yy * 32 + 32, :]

    r2 = layer(a2e, a2o, w2, b2, 16, 48)
    for b in range(_B):
        for yy in range(3):
            a3e[b, pl.ds(yy * 8, 8), :] = r2[b][yy * 16:yy * 16 + 8, :]
            a3o[b, pl.ds(yy * 8, 8), :] = r2[b][yy * 16 + 8:yy * 16 + 16, :]

    r3 = layer(a3e, a3o, w3, b3, 8, 8)
    for b in range(_B):
        r = r3[b].astype(f32)
        out_ref[b, 0, :, :] = r[0:2, :]
        out_ref[b, 1, :, :] = r[4:6, :]


def kernel(x, w0, b0, w1, b1, w2, b2, w3, b3):
    N, Cin, H, W = x.shape
    bf16 = jnp.bfloat16

    xs = x.reshape(N, Cin, 32, 2, 32, 2)
    xs = jnp.transpose(xs, (0, 2, 4, 3, 5, 1)).reshape(N, 1024, 2 * 2 * Cin)
    xs = jnp.pad(xs, ((0, 0), (0, 8), (0, 0))).astype(bf16)

    co0 = w0.shape[0]
    w0t = jnp.transpose(w0, (2, 3, 1, 0)).reshape(2, 2, 2, 2, Cin, co0)
    w0t = jnp.transpose(w0t, (0, 2, 1, 3, 4, 5)).reshape(4, 2 * 2 * Cin, co0)
    w0t = w0t.astype(bf16)

    def prep(w):
        co, ci, kh, kw = w.shape
        return jnp.transpose(w, (2, 3, 1, 0)).reshape(kh * kw, ci, co).astype(bf16)

    w1t, w2t, w3t = prep(w1), prep(w2), prep(w3)
    b0r = b0.reshape(1, co0)
    b1r = b1.reshape(1, w1.shape[0])
    b2r = b2.reshape(1, w2.shape[0])
    b3r = b3.reshape(1, w3.shape[0])

    rows = jax.lax.broadcasted_iota(jnp.int32, (224, 448), 0)
    cols = jax.lax.broadcasted_iota(jnp.int32, (224, 448), 1)
    sel = (cols == 2 * rows).astype(bf16)

    def bcast(op):
        return pl.BlockSpec(op.shape, lambda n, _nd=len(op.shape): (0,) * _nd)

    operands = [xs, w0t, b0r, w1t, b1r, w2t, b2r, w3t, b3r, sel]
    in_specs = [pl.BlockSpec((_B, 1032, 2 * 2 * Cin), lambda n: (n, 0, 0))]
    in_specs += [bcast(op) for op in operands[1:]]

    co3 = w3.shape[0]
    out = pl.pallas_call(
        _kernel_body,
        out_shape=jax.ShapeDtypeStruct((N, 2, 2, co3), jnp.float32),
        grid_spec=pltpu.PrefetchScalarGridSpec(
            num_scalar_prefetch=0,
            grid=(N // _B,),
            in_specs=in_specs,
            out_specs=pl.BlockSpec((_B, 2, 2, co3), lambda n: (n, 0, 0, 0)),
            scratch_shapes=[
                pltpu.VMEM((_B, 512, 32), bf16),
                pltpu.VMEM((_B, 488, 32), bf16),
                pltpu.VMEM((_B, 120, 64), bf16),
                pltpu.VMEM((_B, 120, 64), bf16),
                pltpu.VMEM((_B, 32, 128), bf16),
                pltpu.VMEM((_B, 32, 128), bf16),
            ]),
        compiler_params=pltpu.CompilerParams(dimension_semantics=("parallel",)),
        cost_estimate=pl.CostEstimate(
            flops=2 * N * (992 * 4 * 12 * 32 + 448 * 16 * 32 * 64
                           + 96 * 16 * 64 * 128 + 16 * 16 * 128 * 256),
            transcendentals=0,
            bytes_accessed=int(xs.size * 2 + N * 2 * 2 * co3 * 4)),
    )(*operands)

    return jnp.transpose(out, (0, 3, 1, 2))
